# Initial kernel scaffold; baseline (speedup 1.0000x reference)
import functools

import jax
import jax.numpy as jnp
from jax import lax
from jax.experimental import pallas as pl
from jax.experimental.pallas import tpu as pltpu
from jax.experimental.pallas import tpu_sc as plsc

DIM = 32
NC = 2
NS = 16
NW = NC * NS
GRP = 128
LANES = 16


def _body(job_idx, geek_idx, job_w, geek_w, out_hbm,
          jidx_v, gidx_v, jrows, grows, out_v, sem):
    n_grp = jidx_v.shape[0]
    wid = lax.axis_index("s") * NC + lax.axis_index("c")
    base_g = wid * n_grp

    pltpu.sync_copy(job_idx.at[pl.ds(base_g, n_grp)], jidx_v)
    pltpu.sync_copy(geek_idx.at[pl.ds(base_g, n_grp)], gidx_v)

    lanes = lax.iota(jnp.int32, LANES)

    def group(g, carry):
        cj = pltpu.async_copy(job_w.at[jidx_v.at[g]], jrows, sem)
        cg = pltpu.async_copy(geek_w.at[gidx_v.at[g]], grows, sem)
        cj.wait()
        cg.wait()
        for t in range(GRP // LANES):
            row = lanes + t * LANES
            acc = jnp.zeros((LANES,), jnp.float32)
            for d in range(DIM):
                col = jnp.full((LANES,), d, jnp.int32)
                jv = plsc.load_gather(jrows, [row, col])
                gv = plsc.load_gather(grows, [row, col])
                diff = jv - gv
                acc = acc + diff * diff
            out_v[pl.ds(g * GRP + t * LANES, LANES)] = jnp.sqrt(acc)
        return carry

    lax.fori_loop(0, n_grp, group, 0)
    pltpu.sync_copy(out_v, out_hbm.at[pl.ds(base_g * GRP, n_grp * GRP)])


@jax.jit
def _cml_norm(job_idx, geek_idx, job_w, geek_w):
    n = job_idx.shape[0] * job_idx.shape[1]
    n_grp = n // (NW * GRP)
    mesh = plsc.VectorSubcoreMesh(core_axis_name="c", subcore_axis_name="s")
    return pl.kernel(
        _body,
        out_type=jax.ShapeDtypeStruct((n,), jnp.float32),
        mesh=mesh,
        scratch_types=[
            pltpu.VMEM((n_grp, GRP), jnp.int32),
            pltpu.VMEM((n_grp, GRP), jnp.int32),
            pltpu.VMEM((GRP, DIM), jnp.float32),
            pltpu.VMEM((GRP, DIM), jnp.float32),
            pltpu.VMEM((n_grp * GRP,), jnp.float32),
            pltpu.SemaphoreType.DMA,
        ],
    )(job_idx, geek_idx, job_w, geek_w)


def kernel(job, geek, job_emb_w, geek_emb_w):
    B, L = job.shape
    n = B * L
    jf = job.astype(jnp.int32).reshape(n // GRP, GRP)
    gf = geek.astype(jnp.int32).reshape(n // GRP, GRP)
    out = _cml_norm(jf, gf, job_emb_w, geek_emb_w)
    return out.reshape(B, L)


# speedup vs baseline: 1.1684x; 1.1684x over previous
import functools

import jax
import jax.numpy as jnp
from jax import lax
from jax.experimental import pallas as pl
from jax.experimental.pallas import tpu as pltpu
from jax.experimental.pallas import tpu_sc as plsc

DIM = 32
NC = 2
NS = 16
NW = NC * NS
GRP = 128
LANES = 16


def _body(job_idx, geek_idx, job_w, geek_w, out_hbm,
          jidx_v, gidx_v, jrows, grows, out_v, sem):
    n_grp = jidx_v.shape[0]
    wid = lax.axis_index("s") * NC + lax.axis_index("c")
    base_g = wid * n_grp

    pltpu.sync_copy(job_idx.at[pl.ds(base_g, n_grp)], jidx_v)
    pltpu.sync_copy(geek_idx.at[pl.ds(base_g, n_grp)], gidx_v)

    lanes = lax.iota(jnp.int32, LANES)

    def _sqrt(s):
        i = plsc.bitcast(s, jnp.int32)
        y = plsc.bitcast(
            lax.shift_right_logical(i, 1) + jnp.int32(0x1FBD1DF5), jnp.float32)
        for _ in range(3):
            y = 0.5 * (y + s / y)
        return y

    def group(g, carry):
        cj = pltpu.async_copy(job_w.at[jidx_v.at[g]], jrows, sem)
        cg = pltpu.async_copy(geek_w.at[gidx_v.at[g]], grows, sem)
        cj.wait()
        cg.wait()
        for t in range(GRP // LANES):
            row = lanes + t * LANES
            acc = jnp.zeros((LANES,), jnp.float32)
            for d in range(DIM):
                col = jnp.full((LANES,), d, jnp.int32)
                jv = plsc.load_gather(jrows, [row, col])
                gv = plsc.load_gather(grows, [row, col])
                diff = jv - gv
                acc = acc + diff * diff
            out_v[pl.ds(g * GRP + t * LANES, LANES)] = _sqrt(acc)
        return carry

    lax.fori_loop(0, n_grp, group, 0)
    pltpu.sync_copy(out_v, out_hbm.at[pl.ds(base_g * GRP, n_grp * GRP)])


@jax.jit
def _cml_norm(job_idx, geek_idx, job_w, geek_w):
    n = job_idx.shape[0] * job_idx.shape[1]
    n_grp = n // (NW * GRP)
    mesh = plsc.VectorSubcoreMesh(core_axis_name="c", subcore_axis_name="s")
    return pl.kernel(
        _body,
        out_type=jax.ShapeDtypeStruct((n,), jnp.float32),
        mesh=mesh,
        compiler_params=pltpu.CompilerParams(
            needs_layout_passes=False, use_tc_tiling_on_sc=False),
        scratch_types=[
            pltpu.VMEM((n_grp, GRP), jnp.int32),
            pltpu.VMEM((n_grp, GRP), jnp.int32),
            pltpu.VMEM((GRP, DIM), jnp.float32),
            pltpu.VMEM((GRP, DIM), jnp.float32),
            pltpu.VMEM((n_grp * GRP,), jnp.float32),
            pltpu.SemaphoreType.DMA,
        ],
    )(job_idx, geek_idx, job_w, geek_w)


def kernel(job, geek, job_emb_w, geek_emb_w):
    B, L = job.shape
    n = B * L
    jf = job.astype(jnp.int32).reshape(n // GRP, GRP)
    gf = geek.astype(jnp.int32).reshape(n // GRP, GRP)
    out = _cml_norm(jf, gf, job_emb_w, geek_emb_w)
    return out.reshape(B, L)


# speedup vs baseline: 1.2360x; 1.0579x over previous
import jax
import jax.numpy as jnp
from jax import lax
from jax.experimental import pallas as pl
from jax.experimental.pallas import tpu as pltpu
from jax.experimental.pallas import tpu_sc as plsc

DIM = 32
NC = 2
NS = 16
NW = NC * NS
GRP = 128
SB = 4
ROWS = GRP * SB
LANES = 16


def _body(job_idx, geek_idx, job_w, geek_w, out_hbm,
          jidx_v, gidx_v, jr0, gr0, jr1, gr1, out_v, sem0, sem1):
    n_grp = jidx_v.shape[0]
    n_sb = n_grp // SB
    wid = lax.axis_index("s") * NC + lax.axis_index("c")
    base_g = wid * n_grp

    pltpu.sync_copy(job_idx.at[pl.ds(base_g, n_grp)], jidx_v)
    pltpu.sync_copy(geek_idx.at[pl.ds(base_g, n_grp)], gidx_v)

    lanes = lax.iota(jnp.int32, LANES)

    def _sqrt(s):
        i = plsc.bitcast(s, jnp.int32)
        y = plsc.bitcast(
            lax.shift_right_logical(i, 1) + jnp.int32(0x1FBD1DF5), jnp.float32)
        for _ in range(3):
            y = 0.5 * (y + s / y)
        return y

    def fire(sb, jr, gr, sem):
        for q in range(SB):
            pltpu.async_copy(job_w.at[jidx_v.at[sb * SB + q]],
                             jr.at[pl.ds(q * GRP, GRP)], sem)
            pltpu.async_copy(geek_w.at[gidx_v.at[sb * SB + q]],
                             gr.at[pl.ds(q * GRP, GRP)], sem)

    def drain(jr, gr, sem):
        pltpu.make_async_copy(job_w.at[pl.ds(0, ROWS)], jr, sem).wait()
        pltpu.make_async_copy(geek_w.at[pl.ds(0, ROWS)], gr, sem).wait()

    def compute(sb, jr, gr):
        def block(t, carry):
            row = lanes + t * LANES
            acc = jnp.zeros((LANES,), jnp.float32)
            for d in range(DIM):
                col = jnp.full((LANES,), d, jnp.int32)
                jv = plsc.load_gather(jr, [row, col])
                gv = plsc.load_gather(gr, [row, col])
                diff = jv - gv
                acc = acc + diff * diff
            out_v[pl.ds(sb * ROWS + t * LANES, LANES)] = _sqrt(acc)
            return carry
        lax.fori_loop(0, ROWS // LANES, block, 0)

    fire(0, jr0, gr0, sem0)

    def pair(i, carry):
        fire(2 * i + 1, jr1, gr1, sem1)
        drain(jr0, gr0, sem0)
        compute(2 * i, jr0, gr0)

        @pl.when(i < n_sb // 2 - 1)
        def _():
            fire(2 * i + 2, jr0, gr0, sem0)

        drain(jr1, gr1, sem1)
        compute(2 * i + 1, jr1, gr1)
        return carry

    lax.fori_loop(0, n_sb // 2, pair, 0)
    pltpu.sync_copy(out_v, out_hbm.at[pl.ds(base_g * GRP, n_grp * GRP)])


@jax.jit
def _cml_norm(job_idx, geek_idx, job_w, geek_w):
    n = job_idx.shape[0] * job_idx.shape[1]
    n_grp = n // (NW * GRP)
    mesh = plsc.VectorSubcoreMesh(core_axis_name="c", subcore_axis_name="s")
    return pl.kernel(
        _body,
        out_type=jax.ShapeDtypeStruct((n,), jnp.float32),
        mesh=mesh,
        compiler_params=pltpu.CompilerParams(
            needs_layout_passes=False, use_tc_tiling_on_sc=False),
        scratch_types=[
            pltpu.VMEM((n_grp, GRP), jnp.int32),
            pltpu.VMEM((n_grp, GRP), jnp.int32),
            pltpu.VMEM((ROWS, DIM), jnp.float32),
            pltpu.VMEM((ROWS, DIM), jnp.float32),
            pltpu.VMEM((ROWS, DIM), jnp.float32),
            pltpu.VMEM((ROWS, DIM), jnp.float32),
            pltpu.VMEM((n_grp * GRP,), jnp.float32),
            pltpu.SemaphoreType.DMA,
            pltpu.SemaphoreType.DMA,
        ],
    )(job_idx, geek_idx, job_w, geek_w)


def kernel(job, geek, job_emb_w, geek_emb_w):
    B, L = job.shape
    n = B * L
    jf = job.astype(jnp.int32).reshape(n // GRP, GRP)
    gf = geek.astype(jnp.int32).reshape(n // GRP, GRP)
    out = _cml_norm(jf, gf, job_emb_w, geek_emb_w)
    return out.reshape(B, L)


# speedup vs baseline: 1.2394x; 1.0027x over previous
import jax
import jax.numpy as jnp
from jax import lax
from jax.experimental import pallas as pl
from jax.experimental.pallas import tpu as pltpu
from jax.experimental.pallas import tpu_sc as plsc

DIM = 32
PACK = 4
LINE = DIM * PACK
NC = 2
NS = 16
NW = NC * NS
GRP = 128
LANES = 16


def _body(job_idx, geek_idx, job_w, geek_w, out_hbm,
          jidx_v, gidx_v, jline_v, gline_v,
          jr0, gr0, jr1, gr1, out_v, sem0, sem1):
    n_grp = jidx_v.shape[0]
    wid = lax.axis_index("s") * NC + lax.axis_index("c")
    base_g = wid * n_grp

    pltpu.sync_copy(job_idx.at[pl.ds(base_g, n_grp)], jidx_v)
    pltpu.sync_copy(geek_idx.at[pl.ds(base_g, n_grp)], gidx_v)

    lanes = lax.iota(jnp.int32, LANES)

    def shift(i, carry):
        g, v = i // (GRP // LANES), i % (GRP // LANES)
        s = pl.ds(v * LANES, LANES)
        jline_v[g, s] = lax.shift_right_logical(jidx_v[g, s], 2)
        gline_v[g, s] = lax.shift_right_logical(gidx_v[g, s], 2)
        return carry
    lax.fori_loop(0, n_grp * (GRP // LANES), shift, 0)

    def _sqrt(s):
        i = plsc.bitcast(s, jnp.int32)
        y = plsc.bitcast(
            lax.shift_right_logical(i, 1) + jnp.int32(0x1FBD1DF5), jnp.float32)
        for _ in range(3):
            y = 0.5 * (y + s / y)
        return y

    def fire(g, jr, gr, sem):
        pltpu.async_copy(job_w.at[jline_v.at[g]], jr, sem)
        pltpu.async_copy(geek_w.at[gline_v.at[g]], gr, sem)

    def drain(jr, gr, sem):
        pltpu.make_async_copy(job_w.at[pl.ds(0, GRP)], jr, sem).wait()
        pltpu.make_async_copy(geek_w.at[pl.ds(0, GRP)], gr, sem).wait()

    def compute(g, jr, gr):
        def block(t, carry):
            s = pl.ds(t * LANES, LANES)
            jcol = lax.shift_left(jidx_v[g, s] & 3, 5)
            gcol = lax.shift_left(gidx_v[g, s] & 3, 5)
            row = lanes + t * LANES
            acc = jnp.zeros((LANES,), jnp.float32)
            for d in range(DIM):
                jv = plsc.load_gather(jr, [row, jcol + d])
                gv = plsc.load_gather(gr, [row, gcol + d])
                diff = jv - gv
                acc = acc + diff * diff
            out_v[pl.ds(g * GRP + t * LANES, LANES)] = _sqrt(acc)
            return carry
        lax.fori_loop(0, GRP // LANES, block, 0)

    fire(0, jr0, gr0, sem0)

    def pair(i, carry):
        fire(2 * i + 1, jr1, gr1, sem1)
        drain(jr0, gr0, sem0)
        compute(2 * i, jr0, gr0)

        @pl.when(i < n_grp // 2 - 1)
        def _():
            fire(2 * i + 2, jr0, gr0, sem0)

        drain(jr1, gr1, sem1)
        compute(2 * i + 1, jr1, gr1)
        return carry

    lax.fori_loop(0, n_grp // 2, pair, 0)
    pltpu.sync_copy(out_v, out_hbm.at[pl.ds(base_g * GRP, n_grp * GRP)])


@jax.jit
def _cml_norm(job_idx, geek_idx, job_w, geek_w):
    n = job_idx.shape[0] * job_idx.shape[1]
    n_grp = n // (NW * GRP)
    mesh = plsc.VectorSubcoreMesh(core_axis_name="c", subcore_axis_name="s")
    return pl.kernel(
        _body,
        out_type=jax.ShapeDtypeStruct((n,), jnp.float32),
        mesh=mesh,
        compiler_params=pltpu.CompilerParams(
            needs_layout_passes=False, use_tc_tiling_on_sc=False),
        scratch_types=[
            pltpu.VMEM((n_grp, GRP), jnp.int32),
            pltpu.VMEM((n_grp, GRP), jnp.int32),
            pltpu.VMEM((n_grp, GRP), jnp.int32),
            pltpu.VMEM((n_grp, GRP), jnp.int32),
            pltpu.VMEM((GRP, LINE), jnp.float32),
            pltpu.VMEM((GRP, LINE), jnp.float32),
            pltpu.VMEM((GRP, LINE), jnp.float32),
            pltpu.VMEM((GRP, LINE), jnp.float32),
            pltpu.VMEM((n_grp * GRP,), jnp.float32),
            pltpu.SemaphoreType.DMA,
            pltpu.SemaphoreType.DMA,
        ],
    )(job_idx, geek_idx, job_w, geek_w)


def kernel(job, geek, job_emb_w, geek_emb_w):
    B, L = job.shape
    n = B * L
    jf = job.astype(jnp.int32).reshape(n // GRP, GRP)
    gf = geek.astype(jnp.int32).reshape(n // GRP, GRP)
    jw = job_emb_w.reshape(-1, LINE)
    gw = geek_emb_w.reshape(-1, LINE)
    out = _cml_norm(jf, gf, jw, gw)
    return out.reshape(B, L)


# speedup vs baseline: 1.2422x; 1.0023x over previous
import jax
import jax.numpy as jnp
from jax import lax
from jax.experimental import pallas as pl
from jax.experimental.pallas import tpu as pltpu
from jax.experimental.pallas import tpu_sc as plsc

DIM = 32
NC = 2
NS = 16
NW = NC * NS
GRP = 128
LANES = 16
NBUF = 4


def _body(job_idx, geek_idx, job_w, geek_w, out_hbm,
          jidx_v, gidx_v,
          jr0, gr0, jr1, gr1, jr2, gr2, jr3, gr3,
          out_v, sem0, sem1, sem2, sem3):
    n_per_w = jidx_v.shape[0]
    n_grp = n_per_w // GRP
    wid = lax.axis_index("s") * NC + lax.axis_index("c")
    base = wid * n_per_w
    bufs = ((jr0, gr0, sem0), (jr1, gr1, sem1),
            (jr2, gr2, sem2), (jr3, gr3, sem3))

    pltpu.sync_copy(job_idx.at[pl.ds(base, n_per_w)], jidx_v)
    pltpu.sync_copy(geek_idx.at[pl.ds(base, n_per_w)], gidx_v)

    lanes = lax.iota(jnp.int32, LANES)

    def _sqrt(s):
        i = plsc.bitcast(s, jnp.int32)
        y = plsc.bitcast(
            jnp.int32(0x5F3759DF) - lax.shift_right_logical(i, 1),
            jnp.float32)
        for _ in range(3):
            y = y * (1.5 - 0.5 * s * y * y)
        return s * y

    def fire(g, jr, gr, sem):
        pltpu.async_copy(job_w.at[jidx_v.at[pl.ds(g * GRP, GRP)]], jr, sem)
        pltpu.async_copy(geek_w.at[gidx_v.at[pl.ds(g * GRP, GRP)]], gr, sem)

    def drain(jr, gr, sem):
        pltpu.make_async_copy(job_w.at[pl.ds(0, GRP)], jr, sem).wait()
        pltpu.make_async_copy(geek_w.at[pl.ds(0, GRP)], gr, sem).wait()

    def compute(g, jr, gr):
        def block(t, carry):
            row = lanes + t * LANES
            acc = [jnp.zeros((LANES,), jnp.float32) for _ in range(4)]
            for d in range(DIM):
                col = jnp.full((LANES,), d, jnp.int32)
                jv = plsc.load_gather(jr, [row, col])
                gv = plsc.load_gather(gr, [row, col])
                diff = jv - gv
                acc[d % 4] = acc[d % 4] + diff * diff
            s = (acc[0] + acc[1]) + (acc[2] + acc[3])
            out_v[pl.ds(g * GRP + t * LANES, LANES)] = _sqrt(s)
            return carry
        lax.fori_loop(0, GRP // LANES, block, 0)

    for b in range(NBUF):
        fire(b, *bufs[b])

    def ring(i, carry):
        for b in range(NBUF):
            g = NBUF * i + b
            jr, gr, sem = bufs[b]
            drain(jr, gr, sem)
            compute(g, jr, gr)

            @pl.when(i < n_grp // NBUF - 1)
            def _():
                fire(g + NBUF, jr, gr, sem)
        return carry

    lax.fori_loop(0, n_grp // NBUF, ring, 0)
    pltpu.sync_copy(out_v, out_hbm.at[pl.ds(base, n_per_w)])


@jax.jit
def _cml_norm(job_idx, geek_idx, job_w, geek_w):
    n = job_idx.shape[0]
    n_per_w = n // NW
    mesh = plsc.VectorSubcoreMesh(core_axis_name="c", subcore_axis_name="s")
    rows_t = pltpu.VMEM((GRP, DIM), jnp.float32)
    return pl.kernel(
        _body,
        out_type=jax.ShapeDtypeStruct((n,), jnp.float32),
        mesh=mesh,
        compiler_params=pltpu.CompilerParams(
            needs_layout_passes=False, use_tc_tiling_on_sc=False),
        scratch_types=[
            pltpu.VMEM((n_per_w,), jnp.int32),
            pltpu.VMEM((n_per_w,), jnp.int32),
            rows_t, rows_t, rows_t, rows_t, rows_t, rows_t, rows_t, rows_t,
            pltpu.VMEM((n_per_w,), jnp.float32),
            pltpu.SemaphoreType.DMA,
            pltpu.SemaphoreType.DMA,
            pltpu.SemaphoreType.DMA,
            pltpu.SemaphoreType.DMA,
        ],
    )(job_idx, geek_idx, job_w, geek_w)


def kernel(job, geek, job_emb_w, geek_emb_w):
    B, L = job.shape
    jf = job.astype(jnp.int32).reshape(-1)
    gf = geek.astype(jnp.int32).reshape(-1)
    out = _cml_norm(jf, gf, job_emb_w, geek_emb_w)
    return out.reshape(B, L)


# speedup vs baseline: 1.5795x; 1.2716x over previous
import jax
import jax.numpy as jnp
from jax import lax
from jax.experimental import pallas as pl
from jax.experimental.pallas import tpu as pltpu
from jax.experimental.pallas import tpu_sc as plsc

DIM = 32
NC = 2
NS = 16
NW = NC * NS
GRP = 128
LANES = 16
NBUF = 4


def _body(job_idx, geek_idx, job_w, geek_w, out_hbm,
          jidx_v, gidx_v,
          jr0, gr0, jr1, gr1, jr2, gr2, jr3, gr3,
          out_v, sem0, sem1, sem2, sem3):
    n_per_w = jidx_v.shape[0]
    n_grp = n_per_w // GRP
    wid = lax.axis_index("s") * NC + lax.axis_index("c")
    base = wid * n_per_w
    bufs = ((jr0, gr0, sem0), (jr1, gr1, sem1),
            (jr2, gr2, sem2), (jr3, gr3, sem3))

    pltpu.sync_copy(job_idx.at[pl.ds(base, n_per_w)], jidx_v)
    pltpu.sync_copy(geek_idx.at[pl.ds(base, n_per_w)], gidx_v)

    lanes = lax.iota(jnp.int32, LANES)

    def _sqrt(s):
        i = plsc.bitcast(s, jnp.int32)
        y = plsc.bitcast(
            jnp.int32(0x5F3759DF) - lax.shift_right_logical(i, 1),
            jnp.float32)
        for _ in range(3):
            y = y * (1.5 - 0.5 * s * y * y)
        return s * y

    def fire(g, jr, gr, sem):
        pltpu.async_copy(job_w.at[jidx_v.at[pl.ds(g * GRP, GRP)]], jr, sem)
        pltpu.async_copy(geek_w.at[gidx_v.at[pl.ds(g * GRP, GRP)]], gr, sem)

    def drain(jr, gr, sem):
        pltpu.make_async_copy(job_w.at[pl.ds(0, GRP)], jr, sem).wait()
        pltpu.make_async_copy(geek_w.at[pl.ds(0, GRP)], gr, sem).wait()

    def compute(g, jr, gr):
        def block(t, carry):
            row = lanes + t * LANES
            acc = [jnp.zeros((LANES,), jnp.float32) for _ in range(4)]
            for d in range(DIM):
                col = (lanes + d) & (DIM - 1)
                jv = plsc.load_gather(jr, [row, col])
                gv = plsc.load_gather(gr, [row, col])
                diff = jv - gv
                acc[d % 4] = acc[d % 4] + diff * diff
            s = (acc[0] + acc[1]) + (acc[2] + acc[3])
            out_v[pl.ds(g * GRP + t * LANES, LANES)] = _sqrt(s)
            return carry
        lax.fori_loop(0, GRP // LANES, block, 0)

    for b in range(NBUF):
        fire(b, *bufs[b])

    def ring(i, carry):
        for b in range(NBUF):
            g = NBUF * i + b
            jr, gr, sem = bufs[b]
            drain(jr, gr, sem)
            compute(g, jr, gr)

            @pl.when(i < n_grp // NBUF - 1)
            def _():
                fire(g + NBUF, jr, gr, sem)
        return carry

    lax.fori_loop(0, n_grp // NBUF, ring, 0)
    pltpu.sync_copy(out_v, out_hbm.at[pl.ds(base, n_per_w)])


@jax.jit
def _cml_norm(job_idx, geek_idx, job_w, geek_w):
    n = job_idx.shape[0]
    n_per_w = n // NW
    mesh = plsc.VectorSubcoreMesh(core_axis_name="c", subcore_axis_name="s")
    rows_t = pltpu.VMEM((GRP, DIM), jnp.float32)
    return pl.kernel(
        _body,
        out_type=jax.ShapeDtypeStruct((n,), jnp.float32),
        mesh=mesh,
        compiler_params=pltpu.CompilerParams(
            needs_layout_passes=False, use_tc_tiling_on_sc=False),
        scratch_types=[
            pltpu.VMEM((n_per_w,), jnp.int32),
            pltpu.VMEM((n_per_w,), jnp.int32),
            rows_t, rows_t, rows_t, rows_t, rows_t, rows_t, rows_t, rows_t,
            pltpu.VMEM((n_per_w,), jnp.float32),
            pltpu.SemaphoreType.DMA,
            pltpu.SemaphoreType.DMA,
            pltpu.SemaphoreType.DMA,
            pltpu.SemaphoreType.DMA,
        ],
    )(job_idx, geek_idx, job_w, geek_w)


def kernel(job, geek, job_emb_w, geek_emb_w):
    B, L = job.shape
    jf = job.astype(jnp.int32).reshape(-1)
    gf = geek.astype(jnp.int32).reshape(-1)
    out = _cml_norm(jf, gf, job_emb_w, geek_emb_w)
    return out.reshape(B, L)
